# baseline (speedup 1.0000x reference)
import functools

import jax
import jax.numpy as jnp
from jax import lax
from jax.experimental import pallas as pl
from jax.experimental.pallas import tpu as pltpu
from jax.experimental.pallas import tpu_sc as plsc

_B, _NPTS, _K, _H = 8, 2048, 20, 64
_N = _B * _NPTS
_KP = 24
_TN = 2048
_NCORE, _NSUB = 2, 16
_NW = _NCORE * _NSUB
_RPW = _N // _NW
_CH = 16
_NBLK = _RPW // _CH


def _topk_body(xb_ref, wd_ref, wb_ref, b_ref, idx_ref, q_ref, c_ref, enc_ref,
               *, d, npts, k, kp, tn):
    bb = pl.program_id(0)
    t = pl.program_id(1)
    xa = xb_ref[0]
    xt = xb_ref[0, pl.ds(t * tn, tn), :]
    sq_all = jnp.sum(xa * xa, axis=1)
    sq_t = jnp.sum(xt * xt, axis=1)
    cross = lax.dot_general(xa, xt, (((1,), (1,)), ((), ())),
                            preferred_element_type=jnp.float32)
    dist = sq_all[:, None] + sq_t[None, :] - 2.0 * cross

    q_ref[0] = jnp.dot(xt, wb_ref[:], preferred_element_type=jnp.float32)
    c_ref[0] = jnp.dot(xt, wd_ref[:], preferred_element_type=jnp.float32) + b_ref[0]

    bits = lax.bitcast_convert_type(jnp.maximum(dist, 0.0), jnp.int32)
    iota = lax.broadcasted_iota(jnp.int32, (npts, tn), 0)
    enc_ref[:] = (bits & jnp.int32(-2048)) | iota

    base = bb * npts
    acc0 = base + t * tn + lax.broadcasted_iota(jnp.int32, (kp, tn), 1)
    row_k = lax.broadcasted_iota(jnp.int32, (kp, tn), 0)

    def body(p, carry):
        m_prev, acc = carry
        e = enc_ref[:]
        cand = jnp.where(e > m_prev, e, 2147483647)
        m = jnp.min(cand, axis=0, keepdims=True)
        idxv = (m & jnp.int32(2047)) + base
        acc = jnp.where(row_k == p, idxv, acc)
        return m, acc

    m0 = jnp.full((1, tn), -2147483648, jnp.int32)
    _, acc = lax.fori_loop(0, k, body, (m0, acc0))
    idx_ref[0] = acc


def _tc_layer(x, W, b, d):
    wd = W[:d] - W[d:]
    wb = jnp.zeros((d, 128), jnp.float32).at[:, :_H].set(W[d:])
    xb = x.reshape(_B, _NPTS, d)
    grid = (_B, _NPTS // _TN)
    idx_t, q, c = pl.pallas_call(
        functools.partial(_topk_body, d=d, npts=_NPTS, k=_K, kp=_KP, tn=_TN),
        grid=grid,
        in_specs=[
            pl.BlockSpec((1, _NPTS, d), lambda bb, t: (bb, 0, 0)),
            pl.BlockSpec((d, _H), lambda bb, t: (0, 0)),
            pl.BlockSpec((d, 128), lambda bb, t: (0, 0)),
            pl.BlockSpec((1, _H), lambda bb, t: (0, 0)),
        ],
        out_specs=(
            pl.BlockSpec((1, _KP, _TN), lambda bb, t: (bb, 0, t)),
            pl.BlockSpec((1, _TN, 128), lambda bb, t: (bb, t, 0)),
            pl.BlockSpec((1, _TN, _H), lambda bb, t: (bb, t, 0)),
        ),
        out_shape=(
            jax.ShapeDtypeStruct((_B, _KP, _NPTS), jnp.int32),
            jax.ShapeDtypeStruct((_B, _NPTS, 128), jnp.float32),
            jax.ShapeDtypeStruct((_B, _NPTS, _H), jnp.float32),
        ),
        scratch_shapes=[pltpu.VMEM((_NPTS, _TN), jnp.int32)],
    )(xb, wd, wb, b.reshape(1, _H))
    idx = jnp.transpose(idx_t[:, :_K, :], (0, 2, 1)).reshape(_N * _K)
    return idx, q.reshape(_N, 128), c.reshape(_N, _H)


def _sc_gather_body(q_hbm, c_hbm, idx_hbm, out_hbm, idx_v, rows_v, c_v, out_v,
                    sem):
    wid = lax.axis_index("s") * _NCORE + lax.axis_index("c")
    nvec = _H // 16

    def blk_body(blk, _):
        row0 = wid * _RPW + blk * _CH
        pltpu.sync_copy(idx_hbm.at[pl.ds(row0 * _K, _CH * _K)], idx_v)
        copies = []
        off = 0
        while off < _CH * _K:
            n = min(128, _CH * _K - off)
            copies.append(pltpu.async_copy(
                q_hbm.at[idx_v.at[pl.ds(off, n)]],
                rows_v.at[pl.ds(off, n), :], sem))
            off += n
        pltpu.sync_copy(c_hbm.at[pl.ds(row0, _CH), :], c_v)
        for cp in copies:
            cp.wait()

        def row_body(r, _):
            def k_body(kk, accs):
                return tuple(
                    jnp.maximum(a, rows_v[r * _K + kk, pl.ds(v * 16, 16)])
                    for v, a in enumerate(accs))
            accs = tuple(jnp.full((16,), -jnp.inf, jnp.float32)
                         for _ in range(nvec))
            accs = lax.fori_loop(0, _K, k_body, accs)
            for v in range(nvec):
                out_v[r, pl.ds(v * 16, 16)] = jnp.maximum(
                    accs[v] + c_v[r, pl.ds(v * 16, 16)], 0.0)
            return 0

        lax.fori_loop(0, _CH, row_body, 0)
        pltpu.sync_copy(out_v, out_hbm.at[pl.ds(row0, _CH), :])
        return 0

    lax.fori_loop(0, _NBLK, blk_body, 0)


def _sc_gather_max(q, c, idx):
    mesh = plsc.VectorSubcoreMesh(core_axis_name="c", subcore_axis_name="s",
                                  num_cores=_NCORE, num_subcores=_NSUB)
    f = pl.kernel(
        _sc_gather_body,
        out_type=jax.ShapeDtypeStruct((_N, _H), jnp.float32),
        mesh=mesh,
        scratch_types=[
            pltpu.VMEM((_CH * _K,), jnp.int32),
            pltpu.VMEM((_CH * _K, 128), jnp.float32),
            pltpu.VMEM((_CH, _H), jnp.float32),
            pltpu.VMEM((_CH, _H), jnp.float32),
            pltpu.SemaphoreType.DMA,
        ],
    )
    return f(q, c, idx)


def _mlp_body(x1_ref, x2_ref, x3_ref, wl_ref, bl_ref, wm1_ref, bm1_ref,
              wm2_ref, bm2_ref, wo_ref, bo_ref, out_ref):
    cat = jnp.concatenate([x1_ref[:], x2_ref[:], x3_ref[:]], axis=1)
    h = jnp.maximum(jnp.dot(cat, wl_ref[:], preferred_element_type=jnp.float32)
                    + bl_ref[0], 0.0)
    h = jnp.maximum(jnp.dot(h, wm1_ref[:], preferred_element_type=jnp.float32)
                    + bm1_ref[0], 0.0)
    h = jnp.maximum(jnp.dot(h, wm2_ref[:], preferred_element_type=jnp.float32)
                    + bm2_ref[0], 0.0)
    o = jnp.dot(h, wo_ref[:], preferred_element_type=jnp.float32) + bo_ref[0]
    m = jnp.max(o, axis=1, keepdims=True)
    sh = o - m
    out_ref[:] = sh - jnp.log(jnp.sum(jnp.exp(sh), axis=1, keepdims=True))


def _mlp(x1, x2, x3, Wl, bl, Wm1, bm1, Wm2, bm2, Wo, bo):
    n = x1.shape[0]
    tm = 1024
    nc = Wo.shape[1]
    h = x1.shape[1]
    bw = lambda shape: pl.BlockSpec(shape, lambda i: (0,) * len(shape))
    out = pl.pallas_call(
        _mlp_body,
        grid=(n // tm,),
        in_specs=[
            pl.BlockSpec((tm, h), lambda i: (i, 0)),
            pl.BlockSpec((tm, h), lambda i: (i, 0)),
            pl.BlockSpec((tm, h), lambda i: (i, 0)),
            bw(Wl.shape), bw((1, bl.shape[0])),
            bw(Wm1.shape), bw((1, bm1.shape[0])),
            bw(Wm2.shape), bw((1, bm2.shape[0])),
            bw(Wo.shape), bw((1, bo.shape[0])),
        ],
        out_specs=pl.BlockSpec((tm, nc), lambda i: (i, 0)),
        out_shape=jax.ShapeDtypeStruct((n, nc), jnp.float32),
    )(x1, x2, x3, Wl, bl.reshape(1, -1), Wm1, bm1.reshape(1, -1),
      Wm2, bm2.reshape(1, -1), Wo, bo.reshape(1, -1))
    return out


@jax.jit
def kernel(x, batch, W1, b1, W2, b2, W3, b3, Wl, bl, Wm1, bm1, Wm2, bm2, Wo, bo):
    idx1, q1, c1 = _tc_layer(x, W1, b1, 3)
    x1 = _sc_gather_max(q1, c1, idx1)
    idx2, q2, c2 = _tc_layer(x1, W2, b2, _H)
    x2 = _sc_gather_max(q2, c2, idx2)
    idx3, q3, c3 = _tc_layer(x2, W3, b3, _H)
    x3 = _sc_gather_max(q3, c3, idx3)
    return _mlp(x1, x2, x3, Wl, bl, Wm1, bm1, Wm2, bm2, Wo, bo)


# speedup vs baseline: 1.3046x; 1.3046x over previous
import functools

import jax
import jax.numpy as jnp
from jax import lax
from jax.experimental import pallas as pl
from jax.experimental.pallas import tpu as pltpu
from jax.experimental.pallas import tpu_sc as plsc

_B, _NPTS, _K, _H = 8, 2048, 20, 64
_N = _B * _NPTS
_KP = 24
_TN = 1024
_NCORE, _NSUB = 2, 16
_NW = _NCORE * _NSUB
_RPW = _N // _NW
_CH = 16
_NBLK = _RPW // _CH


def _topk_body(xb_ref, wd_ref, wb_ref, b_ref, idx_ref, q_ref, c_ref, enc_ref,
               *, d, npts, k, kp, tn):
    bb = pl.program_id(0)
    t = pl.program_id(1)
    xa = xb_ref[0]
    xt = xb_ref[0, pl.ds(t * tn, tn), :]
    sq_all = jnp.sum(xa * xa, axis=1)
    sq_t = jnp.sum(xt * xt, axis=1)
    cross = lax.dot_general(xa, xt, (((1,), (1,)), ((), ())),
                            preferred_element_type=jnp.float32)
    dist = sq_all[:, None] + sq_t[None, :] - 2.0 * cross

    q_ref[0] = jnp.dot(xt, wb_ref[:], preferred_element_type=jnp.float32)
    c_ref[0] = jnp.dot(xt, wd_ref[:], preferred_element_type=jnp.float32) + b_ref[0]

    bits = lax.bitcast_convert_type(jnp.maximum(dist, 0.0), jnp.int32)
    iota = lax.broadcasted_iota(jnp.int32, (npts, tn), 0)
    enc_ref[:] = (bits & jnp.int32(-2048)) | iota

    base = bb * npts
    acc0 = base + t * tn + lax.broadcasted_iota(jnp.int32, (kp, tn), 1)
    row_k = lax.broadcasted_iota(jnp.int32, (kp, tn), 0)

    def body(p, carry):
        m_prev, acc = carry
        e = enc_ref[:]
        cand = jnp.where(e > m_prev, e, 2147483647)
        m = jnp.min(cand, axis=0, keepdims=True)
        idxv = (m & jnp.int32(2047)) + base
        acc = jnp.where(row_k == p, idxv, acc)
        return m, acc

    m0 = jnp.full((1, tn), -2147483648, jnp.int32)
    _, acc = lax.fori_loop(0, k, body, (m0, acc0))
    idx_ref[0] = acc


def _tc_layer(x, W, b, d):
    nb = x.shape[0] // _NPTS
    n = nb * _NPTS
    wd = W[:d] - W[d:]
    wb = jnp.zeros((d, 128), jnp.float32).at[:, :_H].set(W[d:])
    xb = x.reshape(nb, _NPTS, d)
    grid = (nb, _NPTS // _TN)
    idx_t, q, c = pl.pallas_call(
        functools.partial(_topk_body, d=d, npts=_NPTS, k=_K, kp=_KP, tn=_TN),
        grid=grid,
        in_specs=[
            pl.BlockSpec((1, _NPTS, d), lambda bb, t: (bb, 0, 0)),
            pl.BlockSpec((d, _H), lambda bb, t: (0, 0)),
            pl.BlockSpec((d, 128), lambda bb, t: (0, 0)),
            pl.BlockSpec((1, _H), lambda bb, t: (0, 0)),
        ],
        out_specs=(
            pl.BlockSpec((1, _KP, _TN), lambda bb, t: (bb, 0, t)),
            pl.BlockSpec((1, _TN, 128), lambda bb, t: (bb, t, 0)),
            pl.BlockSpec((1, _TN, _H), lambda bb, t: (bb, t, 0)),
        ),
        out_shape=(
            jax.ShapeDtypeStruct((nb, _KP, _NPTS), jnp.int32),
            jax.ShapeDtypeStruct((nb, _NPTS, 128), jnp.float32),
            jax.ShapeDtypeStruct((nb, _NPTS, _H), jnp.float32),
        ),
        scratch_shapes=[pltpu.VMEM((_NPTS, _TN), jnp.int32)],
    )(xb, wd, wb, b.reshape(1, _H))
    idx = jnp.transpose(idx_t[:, :_K, :], (0, 2, 1)).reshape(n * _K)
    return idx, q.reshape(n, 128), c.reshape(n, _H)


def _sc_gather_body(q_hbm, c_hbm, idx_hbm, out_hbm, idx_v, rows_v, c_v, out_v,
                    sem, *, rpw, nblk):
    wid = lax.axis_index("s") * _NCORE + lax.axis_index("c")
    nvec = _H // 16

    def blk_body(blk, _):
        row0 = wid * rpw + blk * _CH
        pltpu.sync_copy(idx_hbm.at[pl.ds(row0 * _K, _CH * _K)], idx_v)
        copies = []
        off = 0
        while off < _CH * _K:
            n = min(128, _CH * _K - off)
            copies.append(pltpu.async_copy(
                q_hbm.at[idx_v.at[pl.ds(off, n)]],
                rows_v.at[pl.ds(off, n), :], sem))
            off += n
        pltpu.sync_copy(c_hbm.at[pl.ds(row0, _CH), :], c_v)
        for cp in copies:
            cp.wait()

        def row_body(r, _):
            def k_body(kk, accs):
                return tuple(
                    jnp.maximum(a, rows_v[r * _K + kk, pl.ds(v * 16, 16)])
                    for v, a in enumerate(accs))
            accs = tuple(jnp.full((16,), -jnp.inf, jnp.float32)
                         for _ in range(nvec))
            accs = lax.fori_loop(0, _K, k_body, accs)
            for v in range(nvec):
                out_v[r, pl.ds(v * 16, 16)] = jnp.maximum(
                    accs[v] + c_v[r, pl.ds(v * 16, 16)], 0.0)
            return 0

        lax.fori_loop(0, _CH, row_body, 0)
        pltpu.sync_copy(out_v, out_hbm.at[pl.ds(row0, _CH), :])
        return 0

    lax.fori_loop(0, nblk, blk_body, 0)


def _sc_gather_max(q, c, idx):
    n = q.shape[0]
    rpw = n // _NW
    mesh = plsc.VectorSubcoreMesh(core_axis_name="c", subcore_axis_name="s",
                                  num_cores=_NCORE, num_subcores=_NSUB)
    f = pl.kernel(
        functools.partial(_sc_gather_body, rpw=rpw, nblk=rpw // _CH),
        out_type=jax.ShapeDtypeStruct((n, _H), jnp.float32),
        mesh=mesh,
        scratch_types=[
            pltpu.VMEM((_CH * _K,), jnp.int32),
            pltpu.VMEM((_CH * _K, 128), jnp.float32),
            pltpu.VMEM((_CH, _H), jnp.float32),
            pltpu.VMEM((_CH, _H), jnp.float32),
            pltpu.SemaphoreType.DMA,
        ],
    )
    return f(q, c, idx)


def _mlp_body(x1_ref, x2_ref, x3_ref, wl_ref, bl_ref, wm1_ref, bm1_ref,
              wm2_ref, bm2_ref, wo_ref, bo_ref, out_ref):
    cat = jnp.concatenate([x1_ref[:], x2_ref[:], x3_ref[:]], axis=1)
    h = jnp.maximum(jnp.dot(cat, wl_ref[:], preferred_element_type=jnp.float32)
                    + bl_ref[0], 0.0)
    h = jnp.maximum(jnp.dot(h, wm1_ref[:], preferred_element_type=jnp.float32)
                    + bm1_ref[0], 0.0)
    h = jnp.maximum(jnp.dot(h, wm2_ref[:], preferred_element_type=jnp.float32)
                    + bm2_ref[0], 0.0)
    o = jnp.dot(h, wo_ref[:], preferred_element_type=jnp.float32) + bo_ref[0]
    m = jnp.max(o, axis=1, keepdims=True)
    sh = o - m
    out_ref[:] = sh - jnp.log(jnp.sum(jnp.exp(sh), axis=1, keepdims=True))


def _mlp(x1, x2, x3, Wl, bl, Wm1, bm1, Wm2, bm2, Wo, bo):
    n = x1.shape[0]
    tm = 1024
    nc = Wo.shape[1]
    h = x1.shape[1]
    bw = lambda shape: pl.BlockSpec(shape, lambda i: (0,) * len(shape))
    out = pl.pallas_call(
        _mlp_body,
        grid=(n // tm,),
        in_specs=[
            pl.BlockSpec((tm, h), lambda i: (i, 0)),
            pl.BlockSpec((tm, h), lambda i: (i, 0)),
            pl.BlockSpec((tm, h), lambda i: (i, 0)),
            bw(Wl.shape), bw((1, bl.shape[0])),
            bw(Wm1.shape), bw((1, bm1.shape[0])),
            bw(Wm2.shape), bw((1, bm2.shape[0])),
            bw(Wo.shape), bw((1, bo.shape[0])),
        ],
        out_specs=pl.BlockSpec((tm, nc), lambda i: (i, 0)),
        out_shape=jax.ShapeDtypeStruct((n, nc), jnp.float32),
    )(x1, x2, x3, Wl, bl.reshape(1, -1), Wm1, bm1.reshape(1, -1),
      Wm2, bm2.reshape(1, -1), Wo, bo.reshape(1, -1))
    return out


def _edge_layer(x, W, b, d):
    idx, q, c = _tc_layer(x, W, b, d)
    return _sc_gather_max(q, c, idx)


@jax.jit
def kernel(x, batch, W1, b1, W2, b2, W3, b3, Wl, bl, Wm1, bm1, Wm2, bm2, Wo, bo):
    halves = [x[:_N // 2], x[_N // 2:]]
    outs = []
    for xh in halves:
        x1 = _edge_layer(xh, W1, b1, 3)
        x2 = _edge_layer(x1, W2, b2, _H)
        x3 = _edge_layer(x2, W3, b3, _H)
        outs.append((x1, x2, x3))
    x1 = jnp.concatenate([outs[0][0], outs[1][0]])
    x2 = jnp.concatenate([outs[0][1], outs[1][1]])
    x3 = jnp.concatenate([outs[0][2], outs[1][2]])
    return _mlp(x1, x2, x3, Wl, bl, Wm1, bm1, Wm2, bm2, Wo, bo)


# speedup vs baseline: 1.3272x; 1.0173x over previous
import functools

import jax
import jax.numpy as jnp
from jax import lax
from jax.experimental import pallas as pl
from jax.experimental.pallas import tpu as pltpu
from jax.experimental.pallas import tpu_sc as plsc

_B, _NPTS, _K, _H = 8, 2048, 20, 64
_N = _B * _NPTS
_KP = 24
_TN = 1024
_NCORE, _NSUB = 2, 16
_NW = _NCORE * _NSUB
_RPW = _N // _NW
_CH = 16
_NBLK = _RPW // _CH


def _topk_body(xb_ref, wd_ref, wb_ref, b_ref, idx_ref, q_ref, c_ref, enc_ref,
               *, d, npts, k, kp, tn):
    bb = pl.program_id(0)
    t = pl.program_id(1)
    xa = xb_ref[0]
    xt = xb_ref[0, pl.ds(t * tn, tn), :]
    sq_all = jnp.sum(xa * xa, axis=1)
    sq_t = jnp.sum(xt * xt, axis=1)
    cross = lax.dot_general(xa, xt, (((1,), (1,)), ((), ())),
                            preferred_element_type=jnp.float32)
    dist = sq_all[:, None] + sq_t[None, :] - 2.0 * cross

    q_ref[0] = jnp.dot(xt, wb_ref[:], preferred_element_type=jnp.float32)
    c_ref[0] = jnp.dot(xt, wd_ref[:], preferred_element_type=jnp.float32) + b_ref[0]

    bits = lax.bitcast_convert_type(jnp.maximum(dist, 0.0), jnp.int32)
    iota = lax.broadcasted_iota(jnp.int32, (npts, tn), 0)
    enc_ref[:] = (bits & jnp.int32(-2048)) | iota

    base = bb * npts
    acc0 = base + t * tn + lax.broadcasted_iota(jnp.int32, (kp, tn), 1)
    row_k = lax.broadcasted_iota(jnp.int32, (kp, tn), 0)

    def body(p, carry):
        m_prev, acc = carry
        e = enc_ref[:]
        cand = jnp.where(e > m_prev, e, 2147483647)
        m = jnp.min(cand, axis=0, keepdims=True)
        idxv = (m & jnp.int32(2047)) + base
        acc = jnp.where(row_k == p, idxv, acc)
        return m, acc

    m0 = jnp.full((1, tn), -2147483648, jnp.int32)
    _, acc = lax.fori_loop(0, k, body, (m0, acc0))
    idx_ref[0] = acc


def _tc_layer(x, W, b, d):
    nb = x.shape[0] // _NPTS
    n = nb * _NPTS
    wd = W[:d] - W[d:]
    wb = jnp.zeros((d, 128), jnp.float32).at[:, :_H].set(W[d:])
    xb = x.reshape(nb, _NPTS, d)
    grid = (nb, _NPTS // _TN)
    idx_t, q, c = pl.pallas_call(
        functools.partial(_topk_body, d=d, npts=_NPTS, k=_K, kp=_KP, tn=_TN),
        grid=grid,
        in_specs=[
            pl.BlockSpec((1, _NPTS, d), lambda bb, t: (bb, 0, 0)),
            pl.BlockSpec((d, _H), lambda bb, t: (0, 0)),
            pl.BlockSpec((d, 128), lambda bb, t: (0, 0)),
            pl.BlockSpec((1, _H), lambda bb, t: (0, 0)),
        ],
        out_specs=(
            pl.BlockSpec((1, _KP, _TN), lambda bb, t: (bb, 0, t)),
            pl.BlockSpec((1, _TN, 128), lambda bb, t: (bb, t, 0)),
            pl.BlockSpec((1, _TN, _H), lambda bb, t: (bb, t, 0)),
        ),
        out_shape=(
            jax.ShapeDtypeStruct((nb, _KP, _NPTS), jnp.int32),
            jax.ShapeDtypeStruct((nb, _NPTS, 128), jnp.float32),
            jax.ShapeDtypeStruct((nb, _NPTS, _H), jnp.float32),
        ),
        scratch_shapes=[pltpu.VMEM((_NPTS, _TN), jnp.int32)],
    )(xb, wd, wb, b.reshape(1, _H))
    idx = jnp.transpose(idx_t[:, :_K, :], (0, 2, 1)).reshape(n * _K)
    return idx, q.reshape(n, 128), c.reshape(n, _H)


def _sc_gather_body(q_hbm, c_hbm, idx_hbm, out_hbm, idx_v0, idx_v1, rows_v0,
                    rows_v1, c_v0, c_v1, out_v, sem0, sem1, *, rpw, nblk):
    wid = lax.axis_index("s") * _NCORE + lax.axis_index("c")
    nvec = _H // 16
    idx_vs, rows_vs, c_vs = (idx_v0, idx_v1), (rows_v0, rows_v1), (c_v0, c_v1)
    sems = (sem0, sem1)

    def stage(blk, par):
        row0 = wid * rpw + blk * _CH
        pltpu.sync_copy(idx_hbm.at[pl.ds(row0 * _K, _CH * _K)], idx_vs[par])
        copies = []
        off = 0
        while off < _CH * _K:
            n = min(128, _CH * _K - off)
            copies.append(pltpu.async_copy(
                q_hbm.at[idx_vs[par].at[pl.ds(off, n)]],
                rows_vs[par].at[pl.ds(off, n), :], sems[par]))
            off += n
        copies.append(pltpu.async_copy(
            c_hbm.at[pl.ds(row0, _CH), :], c_vs[par], sems[par]))
        return copies

    def compute(blk, par):
        rows_v, c_v = rows_vs[par], c_vs[par]
        row0 = wid * rpw + blk * _CH

        def row_body(r, _):
            def k_body(kk, accs):
                return tuple(
                    jnp.maximum(a, rows_v[r * _K + kk, pl.ds(v * 16, 16)])
                    for v, a in enumerate(accs))
            accs = tuple(jnp.full((16,), -jnp.inf, jnp.float32)
                         for _ in range(nvec))
            accs = lax.fori_loop(0, _K, k_body, accs)
            for v in range(nvec):
                out_v[r, pl.ds(v * 16, 16)] = jnp.maximum(
                    accs[v] + c_v[r, pl.ds(v * 16, 16)], 0.0)
            return 0

        lax.fori_loop(0, _CH, row_body, 0)
        pltpu.sync_copy(out_v, out_hbm.at[pl.ds(row0, _CH), :])

    pending = {0: stage(0, 0), 1: None}
    for blk in range(nblk):
        par = blk % 2
        if blk + 1 < nblk:
            pending[1 - par] = stage(blk + 1, 1 - par)
        for cp in pending[par]:
            cp.wait()
        compute(blk, par)


def _sc_gather_max(q, c, idx):
    n = q.shape[0]
    rpw = n // _NW
    mesh = plsc.VectorSubcoreMesh(core_axis_name="c", subcore_axis_name="s",
                                  num_cores=_NCORE, num_subcores=_NSUB)
    f = pl.kernel(
        functools.partial(_sc_gather_body, rpw=rpw, nblk=rpw // _CH),
        out_type=jax.ShapeDtypeStruct((n, _H), jnp.float32),
        mesh=mesh,
        scratch_types=[
            pltpu.VMEM((_CH * _K,), jnp.int32),
            pltpu.VMEM((_CH * _K,), jnp.int32),
            pltpu.VMEM((_CH * _K, 128), jnp.float32),
            pltpu.VMEM((_CH * _K, 128), jnp.float32),
            pltpu.VMEM((_CH, _H), jnp.float32),
            pltpu.VMEM((_CH, _H), jnp.float32),
            pltpu.VMEM((_CH, _H), jnp.float32),
            pltpu.SemaphoreType.DMA,
            pltpu.SemaphoreType.DMA,
        ],
    )
    return f(q, c, idx)


def _mlp_body(x1_ref, x2_ref, x3_ref, wl_ref, bl_ref, wm1_ref, bm1_ref,
              wm2_ref, bm2_ref, wo_ref, bo_ref, out_ref):
    cat = jnp.concatenate([x1_ref[:], x2_ref[:], x3_ref[:]], axis=1)
    h = jnp.maximum(jnp.dot(cat, wl_ref[:], preferred_element_type=jnp.float32)
                    + bl_ref[0], 0.0)
    h = jnp.maximum(jnp.dot(h, wm1_ref[:], preferred_element_type=jnp.float32)
                    + bm1_ref[0], 0.0)
    h = jnp.maximum(jnp.dot(h, wm2_ref[:], preferred_element_type=jnp.float32)
                    + bm2_ref[0], 0.0)
    o = jnp.dot(h, wo_ref[:], preferred_element_type=jnp.float32) + bo_ref[0]
    m = jnp.max(o, axis=1, keepdims=True)
    sh = o - m
    out_ref[:] = sh - jnp.log(jnp.sum(jnp.exp(sh), axis=1, keepdims=True))


def _mlp(x1, x2, x3, Wl, bl, Wm1, bm1, Wm2, bm2, Wo, bo):
    n = x1.shape[0]
    tm = 1024
    nc = Wo.shape[1]
    h = x1.shape[1]
    bw = lambda shape: pl.BlockSpec(shape, lambda i: (0,) * len(shape))
    out = pl.pallas_call(
        _mlp_body,
        grid=(n // tm,),
        in_specs=[
            pl.BlockSpec((tm, h), lambda i: (i, 0)),
            pl.BlockSpec((tm, h), lambda i: (i, 0)),
            pl.BlockSpec((tm, h), lambda i: (i, 0)),
            bw(Wl.shape), bw((1, bl.shape[0])),
            bw(Wm1.shape), bw((1, bm1.shape[0])),
            bw(Wm2.shape), bw((1, bm2.shape[0])),
            bw(Wo.shape), bw((1, bo.shape[0])),
        ],
        out_specs=pl.BlockSpec((tm, nc), lambda i: (i, 0)),
        out_shape=jax.ShapeDtypeStruct((n, nc), jnp.float32),
    )(x1, x2, x3, Wl, bl.reshape(1, -1), Wm1, bm1.reshape(1, -1),
      Wm2, bm2.reshape(1, -1), Wo, bo.reshape(1, -1))
    return out


def _edge_layer(x, W, b, d):
    idx, q, c = _tc_layer(x, W, b, d)
    return _sc_gather_max(q, c, idx)


@jax.jit
def kernel(x, batch, W1, b1, W2, b2, W3, b3, Wl, bl, Wm1, bm1, Wm2, bm2, Wo, bo):
    halves = [x[:_N // 2], x[_N // 2:]]
    outs = []
    for xh in halves:
        x1 = _edge_layer(xh, W1, b1, 3)
        x2 = _edge_layer(x1, W2, b2, _H)
        x3 = _edge_layer(x2, W3, b3, _H)
        outs.append((x1, x2, x3))
    x1 = jnp.concatenate([outs[0][0], outs[1][0]])
    x2 = jnp.concatenate([outs[0][1], outs[1][1]])
    x3 = jnp.concatenate([outs[0][2], outs[1][2]])
    return _mlp(x1, x2, x3, Wl, bl, Wm1, bm1, Wm2, bm2, Wo, bo)
